# baseline (speedup 1.0000x reference)
import functools

import jax
import jax.numpy as jnp
from jax import lax
from jax.experimental import pallas as pl
from jax.experimental.pallas import tpu as pltpu
from jax.experimental.pallas import tpu_sc as plsc

B, T, IN, H = 64, 128, 1024, 4096
SRC = H + IN
BLK = 8192
L = 16
K = 64
NG = H // L
GPB = 8
NGB = NG // GPB
LCAP_BLK = 26


def _rnn_body(xp_hbm, grid_hbm, left_hbm, nlb_hbm, bias_hbm, out_hbm,
              srcp, acc0, acc1, biasv, gridb, leftb, nlbv,
              sem0, sem1, seml):
    c = lax.axis_index("c")
    s = lax.axis_index("s")
    wid = s * 2 + c
    b0 = wid * 2
    b1 = b0 + 1
    sems = (sem0, sem1)

    pltpu.sync_copy(bias_hbm, biasv)
    pltpu.sync_copy(nlb_hbm, nlbv)

    @plsc.parallel_loop(0, H // L, unroll=4)
    def zinit(i):
        srcp[pl.ds(i * L, L)] = jnp.zeros((L,), jnp.int32)

    def start_blk(bi, slot):
        pltpu.async_copy(grid_hbm.at[bi], gridb.at[slot], sems[slot])

    def wait_blk(bi, slot):
        pltpu.make_async_copy(grid_hbm.at[bi], gridb.at[slot],
                              sems[slot]).wait()

    nlb = jnp.max(nlbv[pl.ds(0, L)])
    zv = jnp.zeros((L,), jnp.float32)

    def step(t, carry):
        start_blk(0, 0)
        start_blk(1, 1)
        pltpu.sync_copy(xp_hbm.at[wid, t], srcp.at[pl.ds(H, IN)])

        @plsc.parallel_loop(0, H // L, unroll=4)
        def binit(i):
            bv = biasv[pl.ds(i * L, L)]
            acc0[pl.ds(i * L, L)] = bv
            acc1[pl.ds(i * L, L)] = bv

        def pair(g, cc):
            for slot in range(2):
                bi = g * 2 + slot
                wait_blk(bi, slot)
                rb0 = bi * (GPB * L)

                @plsc.parallel_loop(0, K * GPB, unroll=8)
                def gbody(i):
                    cw = gridb[slot, pl.ds(i * L, L)]
                    colv = jnp.bitwise_and(cw, 8191)
                    fv = plsc.bitcast(
                        jnp.bitwise_and(cw, jnp.int32(-65536)),
                        jnp.float32)
                    gp = plsc.load_gather(srcp, [colv])
                    g0, g1 = plsc.unpack(
                        plsc.bitcast(gp, jnp.bfloat16),
                        format=plsc.PackFormat.INTERLEAVED)
                    base = rb0 + jnp.bitwise_and(i, GPB - 1) * L
                    plsc.addupdate(acc0.at[pl.ds(base, L)], g0 * fv)
                    plsc.addupdate(acc1.at[pl.ds(base, L)], g1 * fv)

                @pl.when(bi + 2 < NGB)
                def _():
                    start_blk(bi + 2, slot)
            return cc

        lax.fori_loop(0, NGB // 2, pair, 0)

        def lblock(bi, cc):
            pltpu.async_copy(left_hbm.at[bi], leftb, seml).wait()

            @plsc.parallel_loop(0, BLK // L, unroll=8)
            def linner(i):
                pw = leftb[0, pl.ds(i * L, L)]
                vv = plsc.bitcast(leftb[1, pl.ds(i * L, L)], jnp.float32)
                colsv = jnp.bitwise_and(pw, 8191)
                rowsv = jnp.right_shift(pw, 13)
                gp = plsc.load_gather(srcp, [colsv])
                g0, g1 = plsc.unpack(
                    plsc.bitcast(gp, jnp.bfloat16),
                    format=plsc.PackFormat.INTERLEAVED)
                plsc.addupdate_scatter(acc0, [rowsv], g0 * vv)
                plsc.addupdate_scatter(acc1, [rowsv], g1 * vv)
            return cc

        lax.fori_loop(0, nlb, lblock, 0)

        @plsc.parallel_loop(0, H // L, unroll=4)
        def finish(i):
            sl = pl.ds(i * L, L)
            z0 = acc0[sl]
            e0 = jnp.exp(z0 + z0)
            h0 = 1.0 - 2.0 / (e0 + 1.0)
            z1 = acc1[sl]
            e1 = jnp.exp(z1 + z1)
            h1 = 1.0 - 2.0 / (e1 + 1.0)
            acc0[sl] = h0
            acc1[sl] = h1
            hp = plsc.pack(h0, h1, format=plsc.PackFormat.INTERLEAVED)
            srcp[sl] = plsc.bitcast(hp, jnp.int32)

        pltpu.sync_copy(acc0, out_hbm.at[b0, t])
        pltpu.sync_copy(acc1, out_hbm.at[b1, t])
        return carry

    lax.fori_loop(0, T, step, 0)


def kernel(x, idx_hh, values_hh, idx_ih, values_ih, bias_hh):
    rows = jnp.concatenate([idx_hh[0], idx_ih[0]])
    cols = jnp.concatenate([idx_hh[1], idx_ih[1] + H])
    vals = jnp.concatenate([values_hh, values_ih])
    nnz = rows.shape[0]

    order = jnp.argsort(rows)
    rs = rows[order]
    cs = cols[order]
    vs = vals[order]
    counts = jnp.bincount(rs, length=H)
    cum = jnp.cumsum(counts) - counts
    rank = (jnp.arange(nnz, dtype=jnp.int32) - cum[rs]).astype(jnp.int32)

    vb16 = jax.lax.bitcast_convert_type(
        vs.astype(jnp.bfloat16), jnp.uint16).astype(jnp.uint32)
    cell = jax.lax.bitcast_convert_type(
        (vb16 << 16) | cs.astype(jnp.uint32), jnp.int32)
    gsize = NG * K * L
    pos = ((rs // (GPB * L)) * (GPB * K * L) + rank * (GPB * L)
           + ((rs // L) % GPB) * L + (rs % L))
    grid = jnp.zeros((gsize,), jnp.int32).at[
        jnp.where(rank < K, pos, gsize)].set(cell, mode="drop")
    gridblocks = grid.reshape(NGB, GPB * K * L)

    lm = rank >= K
    lpos = jnp.cumsum(lm) - 1
    lcap = LCAP_BLK * BLK
    tgt = jnp.where(lm, lpos, lcap).astype(jnp.int32)
    lpk = jnp.zeros((lcap,), jnp.int32).at[tgt].set(
        rs * 8192 + cs, mode="drop")
    lvb = jnp.zeros((lcap,), jnp.int32).at[tgt].set(
        jax.lax.bitcast_convert_type(vs, jnp.int32), mode="drop")
    left = jnp.stack([lpk.reshape(LCAP_BLK, BLK),
                      lvb.reshape(LCAP_BLK, BLK)], axis=1)
    n_left = jnp.sum(lm.astype(jnp.int32))
    nlb_arr = jnp.full((L,), (n_left + BLK - 1) // BLK, dtype=jnp.int32)

    bias = bias_hh[:, 0]

    xb = x.astype(jnp.bfloat16)
    xu = jax.lax.bitcast_convert_type(xb, jnp.uint16).astype(jnp.uint32)
    xp = jax.lax.bitcast_convert_type(
        xu[0::2] | (xu[1::2] << 16), jnp.int32)

    mesh = plsc.VectorSubcoreMesh(core_axis_name="c", subcore_axis_name="s")
    run = pl.kernel(
        _rnn_body,
        out_type=jax.ShapeDtypeStruct((B, T, H), jnp.float32),
        mesh=mesh,
        compiler_params=pltpu.CompilerParams(needs_layout_passes=False),
        scratch_types=[
            pltpu.VMEM((SRC,), jnp.int32),
            pltpu.VMEM((H,), jnp.float32),
            pltpu.VMEM((H,), jnp.float32),
            pltpu.VMEM((H,), jnp.float32),
            pltpu.VMEM((2, GPB * K * L), jnp.int32),
            pltpu.VMEM((2, BLK), jnp.int32),
            pltpu.VMEM((L,), jnp.int32),
            pltpu.SemaphoreType.DMA,
            pltpu.SemaphoreType.DMA,
            pltpu.SemaphoreType.DMA,
        ],
    )
    return run(xp, gridblocks, left, nlb_arr, bias)


# speedup vs baseline: 2.0791x; 2.0791x over previous
import functools

import jax
import jax.numpy as jnp
from jax import lax
from jax.experimental import pallas as pl
from jax.experimental.pallas import tpu as pltpu
from jax.experimental.pallas import tpu_sc as plsc

B, T, IN, H = 64, 128, 1024, 4096
SRC = H + IN
BLK = 8192
L = 16


def _rnn_body(nblk, xp_hbm, stream_hbm, bias_hbm, out_hbm,
              srcp, acc0, acc1, biasv, blkb, sem0, sem1):
    c = lax.axis_index("c")
    s = lax.axis_index("s")
    wid = s * 2 + c
    b0 = wid * 2
    b1 = b0 + 1
    sems = (sem0, sem1)

    pltpu.sync_copy(bias_hbm, biasv)

    @plsc.parallel_loop(0, H // L, unroll=4)
    def zinit(i):
        srcp[pl.ds(i * L, L)] = jnp.zeros((L,), jnp.int32)

    def start_blk(bi, slot):
        pltpu.async_copy(stream_hbm.at[bi], blkb.at[slot], sems[slot])

    def wait_blk(bi, slot):
        pltpu.make_async_copy(stream_hbm.at[bi], blkb.at[slot],
                              sems[slot]).wait()

    def step(t, carry):
        start_blk(0, 0)
        start_blk(1, 1)
        pltpu.sync_copy(xp_hbm.at[wid, t], srcp.at[pl.ds(H, IN)])

        @plsc.parallel_loop(0, H // L, unroll=4)
        def binit(i):
            bv = biasv[pl.ds(i * L, L)]
            acc0[pl.ds(i * L, L)] = bv
            acc1[pl.ds(i * L, L)] = bv

        def pair(g, cc):
            for slot in range(2):
                bi = g * 2 + slot
                wait_blk(bi, slot)

                @plsc.parallel_loop(0, BLK // L, unroll=8)
                def inner(i):
                    pw = blkb[slot, 0, pl.ds(i * L, L)]
                    vv = plsc.bitcast(blkb[slot, 1, pl.ds(i * L, L)],
                                      jnp.float32)
                    colsv = jnp.bitwise_and(pw, 8191)
                    rowsv = jnp.right_shift(pw, 13)
                    gp = plsc.load_gather(srcp, [colsv])
                    g0, g1 = plsc.unpack(
                        plsc.bitcast(gp, jnp.bfloat16),
                        format=plsc.PackFormat.INTERLEAVED)
                    plsc.addupdate_scatter(acc0, [rowsv], g0 * vv)
                    plsc.addupdate_scatter(acc1, [rowsv], g1 * vv)

                @pl.when(bi + 2 < nblk)
                def _():
                    start_blk(bi + 2, slot)
            return cc

        lax.fori_loop(0, nblk // 2, pair, 0)

        @plsc.parallel_loop(0, H // L, unroll=4)
        def finish(i):
            sl = pl.ds(i * L, L)
            z0 = acc0[sl]
            e0 = jnp.exp(z0 + z0)
            h0 = 1.0 - 2.0 / (e0 + 1.0)
            z1 = acc1[sl]
            e1 = jnp.exp(z1 + z1)
            h1 = 1.0 - 2.0 / (e1 + 1.0)
            acc0[sl] = h0
            acc1[sl] = h1
            hp = plsc.pack(h0, h1, format=plsc.PackFormat.INTERLEAVED)
            srcp[sl] = plsc.bitcast(hp, jnp.int32)

        pltpu.sync_copy(acc0, out_hbm.at[b0, t])
        pltpu.sync_copy(acc1, out_hbm.at[b1, t])
        return carry

    lax.fori_loop(0, T, step, 0)


def kernel(x, idx_hh, values_hh, idx_ih, values_ih, bias_hh):
    packed_hh = idx_hh[0] * 8192 + idx_hh[1]
    packed_ih = idx_ih[0] * 8192 + (idx_ih[1] + H)
    packed = jnp.concatenate([packed_hh, packed_ih])
    vals = jnp.concatenate([values_hh, values_ih])
    nnz = packed.shape[0]

    nblk = (nnz + BLK - 1) // BLK
    if nblk % 2:
        nblk += 1
    pad = nblk * BLK - nnz
    packed = jnp.pad(packed, (0, pad))
    vals = jnp.pad(vals, (0, pad))
    valbits = jax.lax.bitcast_convert_type(vals, jnp.int32)
    stream = jnp.stack([packed.reshape(nblk, BLK),
                        valbits.reshape(nblk, BLK)], axis=1)
    bias = bias_hh[:, 0]

    xb = x.astype(jnp.bfloat16)
    xu = jax.lax.bitcast_convert_type(xb, jnp.uint16).astype(jnp.uint32)
    xp = jax.lax.bitcast_convert_type(
        xu[0::2] | (xu[1::2] << 16), jnp.int32)

    mesh = plsc.VectorSubcoreMesh(core_axis_name="c", subcore_axis_name="s")
    run = pl.kernel(
        functools.partial(_rnn_body, nblk),
        out_type=jax.ShapeDtypeStruct((B, T, H), jnp.float32),
        mesh=mesh,
        compiler_params=pltpu.CompilerParams(needs_layout_passes=False),
        scratch_types=[
            pltpu.VMEM((SRC,), jnp.int32),
            pltpu.VMEM((H,), jnp.float32),
            pltpu.VMEM((H,), jnp.float32),
            pltpu.VMEM((H,), jnp.float32),
            pltpu.VMEM((2, 2, BLK), jnp.int32),
            pltpu.SemaphoreType.DMA,
            pltpu.SemaphoreType.DMA,
        ],
    )
    return run(xp, stream, bias)


# speedup vs baseline: 2.1127x; 1.0161x over previous
import functools

import jax
import jax.numpy as jnp
from jax import lax
from jax.experimental import pallas as pl
from jax.experimental.pallas import tpu as pltpu
from jax.experimental.pallas import tpu_sc as plsc

B, T, IN, H = 64, 128, 1024, 4096
SRC = H + IN
BLK = 8192
L = 16


def _rnn_body(nblk, xp_hbm, stream_hbm, bias_hbm, out_hbm,
              srcp, acc0, acc1, biasv, blkb, sem0, sem1):
    c = lax.axis_index("c")
    s = lax.axis_index("s")
    wid = s * 2 + c
    b0 = wid * 2
    b1 = b0 + 1
    sems = (sem0, sem1)

    pltpu.sync_copy(bias_hbm, biasv)

    @plsc.parallel_loop(0, H // L, unroll=4)
    def zinit(i):
        srcp[pl.ds(i * L, L)] = jnp.zeros((L,), jnp.int32)

    def start_blk(bi, slot):
        pltpu.async_copy(stream_hbm.at[bi], blkb.at[slot], sems[slot])

    def wait_blk(bi, slot):
        pltpu.make_async_copy(stream_hbm.at[bi], blkb.at[slot],
                              sems[slot]).wait()

    def step(t, carry):
        start_blk(0, 0)
        start_blk(1, 1)
        pltpu.sync_copy(xp_hbm.at[wid, t], srcp.at[pl.ds(H, IN)])

        @plsc.parallel_loop(0, H // L, unroll=4)
        def binit(i):
            bv = biasv[pl.ds(i * L, L)]
            acc0[pl.ds(i * L, L)] = bv
            acc1[pl.ds(i * L, L)] = bv

        def pair(g, cc):
            for slot in range(2):
                bi = g * 2 + slot
                wait_blk(bi, slot)

                @plsc.parallel_loop(0, BLK // L, unroll=16)
                def inner(i):
                    pw = blkb[slot, 0, pl.ds(i * L, L)]
                    vv = plsc.bitcast(blkb[slot, 1, pl.ds(i * L, L)],
                                      jnp.float32)
                    colsv = jnp.bitwise_and(pw, 8191)
                    rowsv = jnp.right_shift(pw, 13)
                    gp = plsc.load_gather(srcp, [colsv])
                    g0, g1 = plsc.unpack(
                        plsc.bitcast(gp, jnp.bfloat16),
                        format=plsc.PackFormat.INTERLEAVED)
                    plsc.addupdate_scatter(acc0, [rowsv], g0 * vv)
                    plsc.addupdate_scatter(acc1, [rowsv], g1 * vv)

                @pl.when(bi + 2 < nblk)
                def _():
                    start_blk(bi + 2, slot)
            return cc

        lax.fori_loop(0, nblk // 2, pair, 0)

        @plsc.parallel_loop(0, H // L, unroll=4)
        def finish(i):
            sl = pl.ds(i * L, L)
            z0 = acc0[sl]
            e0 = jnp.exp(z0 + z0)
            h0 = 1.0 - 2.0 / (e0 + 1.0)
            z1 = acc1[sl]
            e1 = jnp.exp(z1 + z1)
            h1 = 1.0 - 2.0 / (e1 + 1.0)
            acc0[sl] = h0
            acc1[sl] = h1
            hp = plsc.pack(h0, h1, format=plsc.PackFormat.INTERLEAVED)
            srcp[sl] = plsc.bitcast(hp, jnp.int32)

        pltpu.sync_copy(acc0, out_hbm.at[b0, t])
        pltpu.sync_copy(acc1, out_hbm.at[b1, t])
        return carry

    lax.fori_loop(0, T, step, 0)


def kernel(x, idx_hh, values_hh, idx_ih, values_ih, bias_hh):
    packed_hh = idx_hh[0] * 8192 + idx_hh[1]
    packed_ih = idx_ih[0] * 8192 + (idx_ih[1] + H)
    packed = jnp.concatenate([packed_hh, packed_ih])
    vals = jnp.concatenate([values_hh, values_ih])
    nnz = packed.shape[0]

    nblk = (nnz + BLK - 1) // BLK
    if nblk % 2:
        nblk += 1
    pad = nblk * BLK - nnz
    packed = jnp.pad(packed, (0, pad))
    vals = jnp.pad(vals, (0, pad))
    valbits = jax.lax.bitcast_convert_type(vals, jnp.int32)
    stream = jnp.stack([packed.reshape(nblk, BLK),
                        valbits.reshape(nblk, BLK)], axis=1)
    bias = bias_hh[:, 0]

    xb = x.astype(jnp.bfloat16)
    xu = jax.lax.bitcast_convert_type(xb, jnp.uint16).astype(jnp.uint32)
    xp = jax.lax.bitcast_convert_type(
        xu[0::2] | (xu[1::2] << 16), jnp.int32)

    mesh = plsc.VectorSubcoreMesh(core_axis_name="c", subcore_axis_name="s")
    run = pl.kernel(
        functools.partial(_rnn_body, nblk),
        out_type=jax.ShapeDtypeStruct((B, T, H), jnp.float32),
        mesh=mesh,
        compiler_params=pltpu.CompilerParams(needs_layout_passes=False),
        scratch_types=[
            pltpu.VMEM((SRC,), jnp.int32),
            pltpu.VMEM((H,), jnp.float32),
            pltpu.VMEM((H,), jnp.float32),
            pltpu.VMEM((H,), jnp.float32),
            pltpu.VMEM((2, 2, BLK), jnp.int32),
            pltpu.SemaphoreType.DMA,
            pltpu.SemaphoreType.DMA,
        ],
    )
    return run(xp, stream, bias)
